# Initial kernel scaffold; baseline (speedup 1.0000x reference)
import jax
import jax.numpy as jnp
from jax.experimental import pallas as pl


def kernel(x, embs):
    raise NotImplementedError("write your pallas kernel here")


# speedup vs baseline: 1.8873x; 1.8873x over previous
import functools

import jax
import jax.numpy as jnp
from jax import lax
from jax.experimental import pallas as pl
from jax.experimental.pallas import tpu as pltpu
from jax.experimental.pallas import tpu_sc as plsc

_NC = 2
_NS = 16
_NW = _NC * _NS
_L = 16


def _tc_body(x_ref, embs_ref, idx_ref):
    embs = embs_ref[...]
    e2 = jnp.sum(embs * embs, axis=0, keepdims=True)
    kk = embs.shape[1]
    for b in range(x_ref.shape[0]):
        xb = x_ref[b]
        dots = lax.dot_general(
            xb, embs,
            dimension_numbers=(((0,), (0,)), ((), ())),
            preferred_element_type=jnp.float32,
            precision=lax.Precision.HIGHEST,
        )
        scores = e2 - 2.0 * dots
        m = jnp.min(scores, axis=1, keepdims=True)
        iota = lax.broadcasted_iota(jnp.int32, scores.shape, 1)
        idx_ref[b] = jnp.min(
            jnp.where(scores <= m, iota, kk), axis=1
        ).astype(jnp.int32)


def _tc_scores_argmin(xf, embs):
    b, _, p = xf.shape
    return pl.pallas_call(
        _tc_body,
        out_shape=jax.ShapeDtypeStruct((b, p), jnp.int32),
    )(xf, embs)


def _make_sc_gather(bsz, d, p, k):
    d_per_w = d // _NW
    mesh = plsc.VectorSubcoreMesh(core_axis_name="c", subcore_axis_name="s")

    @functools.partial(
        pl.kernel,
        mesh=mesh,
        out_type=jax.ShapeDtypeStruct((bsz * d * p,), jnp.float32),
        scratch_types=[
            pltpu.VMEM((bsz * p,), jnp.int32),
            pltpu.VMEM((d_per_w * k,), jnp.float32),
            pltpu.VMEM((d_per_w * p,), jnp.float32),
        ],
        compiler_params=pltpu.CompilerParams(needs_layout_passes=False),
    )
    def sc_gather(idx_hbm, embs_hbm, out_hbm, idx_v, rows_v, out_v):
        wid = lax.axis_index("s") * _NC + lax.axis_index("c")
        pltpu.sync_copy(idx_hbm, idx_v)
        pltpu.sync_copy(embs_hbm.at[pl.ds(wid * (d_per_w * k), d_per_w * k)],
                        rows_v)
        for b in range(bsz):
            def chunk(c, carry, b=b):
                iv = idx_v[pl.ds(b * p + c * _L, _L)]
                for j in range(d_per_w):
                    g = plsc.load_gather(rows_v, [iv + (j * k)])
                    out_v[pl.ds(j * p + c * _L, _L)] = g
                return carry

            lax.fori_loop(0, p // _L, chunk, 0)
            pltpu.sync_copy(
                out_v,
                out_hbm.at[pl.ds(b * (d * p) + wid * (d_per_w * p),
                                 d_per_w * p)])

    return sc_gather


def kernel(x, embs):
    bsz, d, h, w = x.shape
    k = embs.shape[1]
    p = h * w
    xf = x.reshape(bsz, d, p)
    idx = _tc_scores_argmin(xf, embs)
    out = _make_sc_gather(bsz, d, p, k)(
        idx.reshape(-1), embs.reshape(-1))
    return out.reshape(bsz, d, h, w)


# speedup vs baseline: 1.9424x; 1.0292x over previous
import functools

import jax
import jax.numpy as jnp
from jax import lax
from jax.experimental import pallas as pl
from jax.experimental import pallas as pl2
from jax.experimental.pallas import tpu as pltpu
from jax.experimental.pallas import tpu_sc as plsc

_NC = 2
_NS = 16
_NW = _NC * _NS
_L = 16


def _tc1_body(x_ref, embs_ref, idx_ref, elin_ref):
    embs = embs_ref[...]
    elin_ref[...] = jnp.reshape(embs, (-1,))
    e2 = jnp.sum(embs * embs, axis=0, keepdims=True)
    kk = embs.shape[1]
    bsz = x_ref.shape[0]
    p = x_ref.shape[2]
    for b in range(bsz):
        xb = x_ref[b]
        dots = lax.dot_general(
            xb, embs,
            dimension_numbers=(((0,), (0,)), ((), ())),
            preferred_element_type=jnp.float32,
            precision=lax.Precision.HIGHEST,
        )
        scores = e2 - 2.0 * dots
        m = jnp.min(scores, axis=1, keepdims=True)
        iota = lax.broadcasted_iota(jnp.int32, scores.shape, 1)
        idx_ref[pl.ds(b * p, p)] = jnp.min(
            jnp.where(scores <= m, iota, kk), axis=1
        ).astype(jnp.int32)


def _tc1(xf, embs):
    bsz, d, p = xf.shape
    k = embs.shape[1]
    return pl.pallas_call(
        _tc1_body,
        out_shape=(
            jax.ShapeDtypeStruct((bsz * p,), jnp.int32),
            jax.ShapeDtypeStruct((d * k,), jnp.float32),
        ),
    )(xf, embs)


def _make_sc_gather(bsz, d, p, k):
    d_per_w = d // _NW
    rows_words = d_per_w * k
    blk_words = d_per_w * p
    n_rows = blk_words // 128
    mesh = plsc.VectorSubcoreMesh(core_axis_name="c", subcore_axis_name="s")

    @functools.partial(
        pl.kernel,
        mesh=mesh,
        out_type=jax.ShapeDtypeStruct((bsz * d * p // 128, 128), jnp.float32),
        scratch_types=[
            pltpu.VMEM((bsz * p,), jnp.int32),
            pltpu.VMEM((rows_words,), jnp.float32),
            pltpu.VMEM((n_rows, 128), jnp.float32),
        ],
        compiler_params=pltpu.CompilerParams(needs_layout_passes=False),
    )
    def sc_gather(idx_hbm, elin_hbm, out_hbm, idx_v, rows_v, out_v):
        wid = lax.axis_index("s") * _NC + lax.axis_index("c")
        pltpu.sync_copy(idx_hbm, idx_v)
        pltpu.sync_copy(elin_hbm.at[pl.ds(wid * rows_words, rows_words)],
                        rows_v)
        for b in range(bsz):
            for r in range(n_rows):
                base = (r // 2) * k
                half = (r % 2) * 128

                def cc_body(cc, carry, b=b, base=base, half=half, r=r):
                    iv = idx_v[pl.ds(b * p + half + cc * _L, _L)]
                    g = plsc.load_gather(rows_v, [iv + base])
                    out_v[r, pl.ds(cc * _L, _L)] = g
                    return carry

                lax.fori_loop(0, 128 // _L, cc_body, 0)
            pltpu.sync_copy(
                out_v,
                out_hbm.at[pl.ds(b * (d * p // 128) + wid * n_rows, n_rows)])

    return sc_gather


def kernel(x, embs):
    bsz, d, h, w = x.shape
    k = embs.shape[1]
    p = h * w
    xf = x.reshape(bsz, d, p)
    idx, elin = _tc1(xf, embs)
    g = _make_sc_gather(bsz, d, p, k)(idx, elin)
    return g.reshape(bsz, d, h, w)


# speedup vs baseline: 2.1322x; 1.0977x over previous
import functools

import jax
import jax.numpy as jnp
from jax import lax
from jax.experimental import pallas as pl
from jax.experimental import pallas as pl2
from jax.experimental.pallas import tpu as pltpu
from jax.experimental.pallas import tpu_sc as plsc

_NC = 2
_NS = 16
_NW = _NC * _NS
_L = 16


def _tc1_body(x_ref, embs_ref, idx_ref, elin_ref):
    embs = embs_ref[...]
    elin_ref[...] = jnp.reshape(embs, (-1,))
    e2 = jnp.sum(embs * embs, axis=0, keepdims=True)
    kk = embs.shape[1]
    bsz = x_ref.shape[0]
    p = x_ref.shape[2]
    for b in range(bsz):
        xb = x_ref[b]
        dots = lax.dot_general(
            xb, embs,
            dimension_numbers=(((0,), (0,)), ((), ())),
            preferred_element_type=jnp.float32,
            precision=lax.Precision.HIGHEST,
        )
        scores = e2 - 2.0 * dots
        m = jnp.min(scores, axis=1, keepdims=True)
        iota = lax.broadcasted_iota(jnp.int32, scores.shape, 1)
        idx_ref[pl.ds(b * p, p)] = jnp.min(
            jnp.where(scores <= m, iota, kk), axis=1
        ).astype(jnp.int32)


def _tc1(xf, embs):
    bsz, d, p = xf.shape
    k = embs.shape[1]
    return pl.pallas_call(
        _tc1_body,
        out_shape=(
            jax.ShapeDtypeStruct((bsz * p,), jnp.int32),
            jax.ShapeDtypeStruct((d * k,), jnp.float32),
        ),
    )(xf, embs)


def _make_sc_gather(bsz, d, p, k):
    d_per_w = d // _NW
    rows_words = d_per_w * k
    blk_words = d_per_w * p
    n_rows = blk_words // 128
    mesh = plsc.VectorSubcoreMesh(core_axis_name="c", subcore_axis_name="s")

    hh = 16
    ww = p // hh

    @functools.partial(
        pl.kernel,
        mesh=mesh,
        out_type=jax.ShapeDtypeStruct((bsz, d, hh, ww), jnp.float32),
        scratch_types=[
            pltpu.VMEM((bsz * p,), jnp.int32),
            pltpu.VMEM((rows_words,), jnp.float32),
            pltpu.VMEM((d_per_w, hh, ww), jnp.float32),
        ],
        compiler_params=pltpu.CompilerParams(needs_layout_passes=False),
    )
    def sc_gather(idx_hbm, elin_hbm, out_hbm, idx_v, rows_v, out_v):
        wid = lax.axis_index("s") * _NC + lax.axis_index("c")
        d0 = wid * d_per_w
        pltpu.sync_copy(idx_hbm, idx_v)
        pltpu.sync_copy(elin_hbm.at[pl.ds(wid * rows_words, rows_words)],
                        rows_v)
        for b in range(bsz):
            for j in range(d_per_w):
                for i in range(hh):
                    iv = idx_v[pl.ds(b * p + i * ww, ww)]
                    g = plsc.load_gather(rows_v, [iv + j * k])
                    out_v[j, i, :] = g
            pltpu.sync_copy(out_v, out_hbm.at[b, pl.ds(d0, d_per_w)])

    return sc_gather


def kernel(x, embs):
    bsz, d, h, w = x.shape
    k = embs.shape[1]
    p = h * w
    xf = x.reshape(bsz, d, p)
    idx, elin = _tc1(xf, embs)
    return _make_sc_gather(bsz, d, p, k)(idx, elin)
